# Initial kernel scaffold; baseline (speedup 1.0000x reference)
import jax
import jax.numpy as jnp
from jax.experimental import pallas as pl


def kernel(x, f_ij, rcut_ij, W_in2f, W_fn1, b_fn1, W_fn2, b_fn2, W_f1, b_f1, W_f2, b_f2):
    raise NotImplementedError("write your pallas kernel here")


# speedup vs baseline: 7.4086x; 7.4086x over previous
import math
import functools

import jax
import jax.numpy as jnp
from jax.experimental import pallas as pl

_LOG2 = math.log(2.0)


def _ssp(v):
    return jnp.maximum(v, 0.0) + jnp.log1p(jnp.exp(-jnp.abs(v))) - _LOG2


def _schnet_body(x_ref, f_ref, rc_ref,
                 w_in2f_t, w_fn1_t, b_fn1, w_fn2_t, b_fn2,
                 w_f1_t, b_f1, w_f2_t, b_f2,
                 out_ref, *, ti, n, r, f_dim):
    y = jnp.dot(x_ref[0], w_in2f_t[...], preferred_element_type=jnp.float32)
    fb = f_ref[0].reshape(ti * n, r)
    h = _ssp(jnp.dot(fb, w_fn1_t[...], preferred_element_type=jnp.float32)
             + b_fn1[...])
    w = (jnp.dot(h, w_fn2_t[...], preferred_element_type=jnp.float32)
         + b_fn2[...])
    w = w.reshape(ti, n, f_dim) * rc_ref[0][:, :, None]
    acc = jnp.sum(w * y[None, :, :], axis=1)
    z = _ssp(jnp.dot(acc, w_f1_t[...], preferred_element_type=jnp.float32)
             + b_f1[...])
    out_ref[0] = (jnp.dot(z, w_f2_t[...], preferred_element_type=jnp.float32)
                  + b_f2[...])


def kernel(x, f_ij, rcut_ij, W_in2f, W_fn1, b_fn1, W_fn2, b_fn2,
           W_f1, b_f1, W_f2, b_f2):
    B, N, A = x.shape
    R = f_ij.shape[-1]
    F = W_in2f.shape[0]
    TI = 32
    grid = (B, N // TI)

    full = lambda arr: pl.BlockSpec(arr.shape, lambda b, i: (0,) * arr.ndim)
    body = functools.partial(_schnet_body, ti=TI, n=N, r=R, f_dim=F)

    wt = dict(
        w_in2f_t=W_in2f.T, w_fn1_t=W_fn1.T, b_fn1=b_fn1.reshape(1, F),
        w_fn2_t=W_fn2.T, b_fn2=b_fn2.reshape(1, F),
        w_f1_t=W_f1.T, b_f1=b_f1.reshape(1, A),
        w_f2_t=W_f2.T, b_f2=b_f2.reshape(1, A),
    )

    out = pl.pallas_call(
        body,
        grid=grid,
        in_specs=[
            pl.BlockSpec((1, N, A), lambda b, i: (b, 0, 0)),
            pl.BlockSpec((1, TI, N, R), lambda b, i: (b, i, 0, 0)),
            pl.BlockSpec((1, TI, N), lambda b, i: (b, i, 0)),
            full(wt["w_in2f_t"]), full(wt["w_fn1_t"]), full(wt["b_fn1"]),
            full(wt["w_fn2_t"]), full(wt["b_fn2"]),
            full(wt["w_f1_t"]), full(wt["b_f1"]),
            full(wt["w_f2_t"]), full(wt["b_f2"]),
        ],
        out_specs=pl.BlockSpec((1, TI, A), lambda b, i: (b, i, 0)),
        out_shape=jax.ShapeDtypeStruct((B, N, A), jnp.float32),
    )(x, f_ij, rcut_ij, *wt.values())
    return out


# speedup vs baseline: 7.5270x; 1.0160x over previous
import math
import functools

import jax
import jax.numpy as jnp
from jax.experimental import pallas as pl

_LOG2 = math.log(2.0)


def _ssp(v):
    return jnp.maximum(v, 0.0) + jnp.log1p(jnp.exp(-jnp.abs(v))) - _LOG2


def _schnet_body(x_ref, f_ref, rc_ref,
                 w_in2f_t, w_fn1_t, b_fn1, w_fn2_t, b_fn2,
                 w_f1_t, b_f1, w_f2_t, b_f2,
                 out_ref, *, ti, n, r, f_dim):
    y = jnp.dot(x_ref[0], w_in2f_t[...], preferred_element_type=jnp.float32)
    fb = f_ref[0].reshape(ti * n, r)
    h = _ssp(jnp.dot(fb, w_fn1_t[...], preferred_element_type=jnp.float32)
             + b_fn1[...])
    w = (jnp.dot(h, w_fn2_t[...], preferred_element_type=jnp.float32)
         + b_fn2[...])
    w = w.reshape(ti, n, f_dim) * rc_ref[0][:, :, None]
    acc = jnp.sum(w * y[None, :, :], axis=1)
    z = _ssp(jnp.dot(acc, w_f1_t[...], preferred_element_type=jnp.float32)
             + b_f1[...])
    out_ref[0] = (jnp.dot(z, w_f2_t[...], preferred_element_type=jnp.float32)
                  + b_f2[...])


def kernel(x, f_ij, rcut_ij, W_in2f, W_fn1, b_fn1, W_fn2, b_fn2,
           W_f1, b_f1, W_f2, b_f2):
    B, N, A = x.shape
    R = f_ij.shape[-1]
    F = W_in2f.shape[0]
    TI = 64
    grid = (B, N // TI)

    full = lambda arr: pl.BlockSpec(arr.shape, lambda b, i: (0,) * arr.ndim)
    body = functools.partial(_schnet_body, ti=TI, n=N, r=R, f_dim=F)

    wt = dict(
        w_in2f_t=W_in2f.T, w_fn1_t=W_fn1.T, b_fn1=b_fn1.reshape(1, F),
        w_fn2_t=W_fn2.T, b_fn2=b_fn2.reshape(1, F),
        w_f1_t=W_f1.T, b_f1=b_f1.reshape(1, A),
        w_f2_t=W_f2.T, b_f2=b_f2.reshape(1, A),
    )

    out = pl.pallas_call(
        body,
        grid=grid,
        in_specs=[
            pl.BlockSpec((1, N, A), lambda b, i: (b, 0, 0)),
            pl.BlockSpec((1, TI, N, R), lambda b, i: (b, i, 0, 0)),
            pl.BlockSpec((1, TI, N), lambda b, i: (b, i, 0)),
            full(wt["w_in2f_t"]), full(wt["w_fn1_t"]), full(wt["b_fn1"]),
            full(wt["w_fn2_t"]), full(wt["b_fn2"]),
            full(wt["w_f1_t"]), full(wt["b_f1"]),
            full(wt["w_f2_t"]), full(wt["b_f2"]),
        ],
        out_specs=pl.BlockSpec((1, TI, A), lambda b, i: (b, i, 0)),
        out_shape=jax.ShapeDtypeStruct((B, N, A), jnp.float32),
    )(x, f_ij, rcut_ij, *wt.values())
    return out
